# baseline (speedup 1.0000x reference)
import functools

import jax
import jax.numpy as jnp
from jax import lax
from jax.experimental import pallas as pl
from jax.experimental.pallas import tpu as pltpu
from jax.experimental.pallas import tpu_sc as plsc

B = 16384
K = 32
D = 128
NC = 2
NS = 16
NW = NC * NS
BPW = B // NW
BLK = 16
NBLK = BPW // BLK
HALF = 16


def _tree(vals, op):
    vals = list(vals)
    while len(vals) > 1:
        vals = [op(vals[i], vals[i + 1]) if i + 1 < len(vals) else vals[i]
                for i in range(0, len(vals), 2)]
    return vals[0]


def _sc_body(node_hbm, hyper_hbm, h_hbm, xr_hbm, out_hbm,
             xidx, hidx, emb, src, outb, tbuf, mrows, mbuf, se0, se1, ss):
    wid = lax.axis_index("s") * NC + lax.axis_index("c")
    base_blk = wid * NBLK
    lanes = lax.iota(jnp.int32, 16)

    pltpu.sync_copy(xr_hbm.at[pl.ds(base_blk * 2, 2 * NBLK)], xidx)
    pltpu.sync_copy(h_hbm.at[pl.ds(wid * BPW, BPW)], hidx)

    def issue_emb(tloc, hf):
        sem = se0 if hf == 0 else se1
        pltpu.async_copy(node_hbm.at[xidx.at[tloc, 0]],
                         emb.at[hf, pl.ds(0, 128)], sem)
        pltpu.async_copy(node_hbm.at[xidx.at[tloc, 1]],
                         emb.at[hf, pl.ds(128, 128)], sem)

    def wait_emb(tloc, hf):
        sem = se0 if hf == 0 else se1
        pltpu.make_async_copy(node_hbm.at[xidx.at[tloc, 0]],
                              emb.at[hf, pl.ds(0, 128)], sem).wait()
        pltpu.make_async_copy(node_hbm.at[xidx.at[tloc, 1]],
                              emb.at[hf, pl.ds(128, 128)], sem).wait()

    def issue_src(blk, pb):
        pltpu.async_copy(hyper_hbm.at[hidx.at[pl.ds(blk * BLK, BLK)]],
                         src.at[pb], ss)

    def wait_src(blk, pb):
        pltpu.make_async_copy(hyper_hbm.at[hidx.at[pl.ds(blk * BLK, BLK)]],
                              src.at[pb], ss).wait()

    def compute_half(hf, pb):
        @pl.loop(0, BLK)
        def cloop(c):
            sv = []
            for i in range(4):
                sv.extend(plsc.unpack(
                    plsc.bitcast(src[pb, c, pl.ds(i * 16, 16)], jnp.bfloat16),
                    format=plsc.PackFormat.INTERLEAVED))
            for j in range(HALF):
                row = c * HALF + j
                p = []
                for i in range(4):
                    ee, eo = plsc.unpack(
                        plsc.bitcast(emb[hf, row, pl.ds(i * 16, 16)],
                                     jnp.bfloat16),
                        format=plsc.PackFormat.INTERLEAVED)
                    p.append(sv[2 * i] * ee)
                    p.append(sv[2 * i + 1] * eo)
                tbuf[j, pl.ds(0, 16)] = _tree(p, lambda a, b: a + b)
            cols = [plsc.load_gather(tbuf, [lanes, jnp.full((16,), l, jnp.int32)])
                    for l in range(16)]
            mrows[c, pl.ds(0, 16)] = _tree(cols, lambda a, b: a + b)
        cols = [plsc.load_gather(mrows, [lanes, jnp.full((16,), l, jnp.int32)])
                for l in range(16)]
        return _tree(cols, jnp.minimum)

    issue_src(0, 0)
    issue_emb(0, 0)

    @pl.loop(0, NBLK // 2)
    def superblock(sb):
        for pb in (0, 1):
            blk = sb * 2 + pb
            tloc = blk * 2
            issue_emb(tloc + 1, 1)
            wait_src(blk, pb)
            wait_emb(tloc, 0)
            mbuf[pl.ds(0, BLK)] = compute_half(0, pb)
            if pb == 0:
                issue_emb(tloc + 2, 0)
                issue_src(blk + 1, 1)
            else:
                @pl.when(sb + 1 < NBLK // 2)
                def _():
                    issue_emb(tloc + 2, 0)
                    issue_src(blk + 1, 0)
            wait_emb(tloc + 1, 1)
            m = jnp.minimum(compute_half(1, pb), mbuf[pl.ds(0, BLK)])
            outb[pl.ds(blk * BLK, BLK)] = m

    pltpu.sync_copy(outb, out_hbm.at[pl.ds(wid * BPW, BPW)])


@jax.jit
def _sc_call(node_emb, hyperedge_emb, h32, xr):
    mesh = plsc.VectorSubcoreMesh(core_axis_name="c", subcore_axis_name="s")
    return pl.kernel(
        _sc_body,
        out_type=jax.ShapeDtypeStruct((B,), jnp.float32),
        mesh=mesh,
        compiler_params=pltpu.CompilerParams(needs_layout_passes=False,
                                             use_tc_tiling_on_sc=False),
        scratch_types=[
            pltpu.VMEM((2 * NBLK, 2, 128), jnp.int32),
            pltpu.VMEM((BPW,), jnp.int32),
            pltpu.VMEM((2, 256, 64), jnp.int32),
            pltpu.VMEM((2, BLK, 64), jnp.int32),
            pltpu.VMEM((BPW,), jnp.float32),
            pltpu.VMEM((HALF, 17), jnp.float32),
            pltpu.VMEM((BLK, 17), jnp.float32),
            pltpu.VMEM((BLK,), jnp.float32),
            pltpu.SemaphoreType.DMA,
            pltpu.SemaphoreType.DMA,
            pltpu.SemaphoreType.DMA,
        ],
    )(node_emb, hyperedge_emb, h32, xr)


def kernel(node_emb, hyperedge_emb, h, X):
    h32 = h.astype(jnp.int32)
    node_emb = lax.bitcast_convert_type(
        node_emb.astype(jnp.bfloat16).reshape(-1, D // 2, 2), jnp.int32)
    hyperedge_emb = lax.bitcast_convert_type(
        hyperedge_emb.astype(jnp.bfloat16).reshape(-1, D // 2, 2), jnp.int32)
    xr = (X.astype(jnp.int32)
          .reshape(B // BLK, BLK, 2, HALF)
          .transpose(0, 2, 1, 3)
          .reshape(B // BLK * 2, 2, 128))
    out = jax.nn.sigmoid(_sc_call(node_emb, hyperedge_emb, h32, xr))
    return out.reshape(B, 1)


# speedup vs baseline: 3.0693x; 3.0693x over previous
import functools

import jax
import jax.numpy as jnp
from jax import lax
from jax.experimental import pallas as pl
from jax.experimental.pallas import tpu as pltpu
from jax.experimental.pallas import tpu_sc as plsc

B = 16384
K = 32
D = 128
NC = 2
NS = 16
NW = NC * NS
BPW = B // NW
BLK = 16
NBLK = BPW // BLK
HALF = 16


def _tree(vals, op):
    vals = list(vals)
    while len(vals) > 1:
        vals = [op(vals[i], vals[i + 1]) if i + 1 < len(vals) else vals[i]
                for i in range(0, len(vals), 2)]
    return vals[0]


def _sc_body(node_hbm, hyper_hbm, h_hbm, xr_hbm, out_hbm,
             xidx, hidx, emb, src, outb, tbuf, mrows, mbuf, se0, se1, ss):
    wid = lax.axis_index("s") * NC + lax.axis_index("c")
    base_blk = wid * NBLK
    lanes = lax.iota(jnp.int32, 16)

    pltpu.sync_copy(xr_hbm.at[pl.ds(base_blk * 2, 2 * NBLK)], xidx)
    pltpu.sync_copy(h_hbm.at[pl.ds(wid * BPW, BPW)], hidx)

    def issue_emb(tloc, hf):
        sem = se0 if hf == 0 else se1
        pltpu.async_copy(node_hbm.at[xidx.at[tloc, 0]],
                         emb.at[hf, pl.ds(0, 128)], sem)
        pltpu.async_copy(node_hbm.at[xidx.at[tloc, 1]],
                         emb.at[hf, pl.ds(128, 128)], sem)

    def wait_emb(tloc, hf):
        sem = se0 if hf == 0 else se1
        pltpu.make_async_copy(node_hbm.at[xidx.at[tloc, 0]],
                              emb.at[hf, pl.ds(0, 128)], sem).wait()
        pltpu.make_async_copy(node_hbm.at[xidx.at[tloc, 1]],
                              emb.at[hf, pl.ds(128, 128)], sem).wait()

    def issue_src(blk, pb):
        pltpu.async_copy(hyper_hbm.at[hidx.at[pl.ds(blk * BLK, BLK)]],
                         src.at[pb], ss)

    def wait_src(blk, pb):
        pltpu.make_async_copy(hyper_hbm.at[hidx.at[pl.ds(blk * BLK, BLK)]],
                              src.at[pb], ss).wait()

    def compute_half(hf, pb):
        @pl.loop(0, BLK)
        def cloop(c):
            sv = [src[pb, c, pl.ds(i * 16, 16)] for i in range(8)]
            accs = []
            for j in range(HALF):
                row = c * HALF + j
                p = [sv[i] * emb[hf, row, pl.ds(i * 16, 16)] for i in range(8)]
                accs.append(_tree(p, lambda a, b: a + b))
            for j in range(HALF):
                tbuf[j, pl.ds(0, 16)] = accs[j]
            cols = [plsc.load_gather(tbuf, [lanes, jnp.full((16,), l, jnp.int32)])
                    for l in range(16)]
            mrows[c, pl.ds(0, 16)] = _tree(cols, lambda a, b: a + b)
        cols = [plsc.load_gather(mrows, [lanes, jnp.full((16,), l, jnp.int32)])
                for l in range(16)]
        return _tree(cols, jnp.minimum)

    issue_src(0, 0)
    issue_emb(0, 0)

    @pl.loop(0, NBLK // 2)
    def superblock(sb):
        for pb in (0, 1):
            blk = sb * 2 + pb
            tloc = blk * 2
            issue_emb(tloc + 1, 1)
            wait_src(blk, pb)
            wait_emb(tloc, 0)
            mbuf[pl.ds(0, BLK)] = compute_half(0, pb)
            if pb == 0:
                issue_emb(tloc + 2, 0)
                issue_src(blk + 1, 1)
            else:
                @pl.when(sb + 1 < NBLK // 2)
                def _():
                    issue_emb(tloc + 2, 0)
                    issue_src(blk + 1, 0)
            wait_emb(tloc + 1, 1)
            m = jnp.minimum(compute_half(1, pb), mbuf[pl.ds(0, BLK)])
            outb[pl.ds(blk * BLK, BLK)] = m

    pltpu.sync_copy(outb, out_hbm.at[pl.ds(wid * BPW, BPW)])


@jax.jit
def _sc_call(node_emb, hyperedge_emb, h32, xr):
    mesh = plsc.VectorSubcoreMesh(core_axis_name="c", subcore_axis_name="s")
    return pl.kernel(
        _sc_body,
        out_type=jax.ShapeDtypeStruct((B,), jnp.float32),
        mesh=mesh,
        compiler_params=pltpu.CompilerParams(needs_layout_passes=False),
        scratch_types=[
            pltpu.VMEM((2 * NBLK, 2, 128), jnp.int32),
            pltpu.VMEM((BPW,), jnp.int32),
            pltpu.VMEM((2, 256, 128), jnp.float32),
            pltpu.VMEM((2, BLK, 128), jnp.float32),
            pltpu.VMEM((BPW,), jnp.float32),
            pltpu.VMEM((HALF, 17), jnp.float32),
            pltpu.VMEM((BLK, 17), jnp.float32),
            pltpu.VMEM((BLK,), jnp.float32),
            pltpu.SemaphoreType.DMA,
            pltpu.SemaphoreType.DMA,
            pltpu.SemaphoreType.DMA,
        ],
    )(node_emb, hyperedge_emb, h32, xr)


def kernel(node_emb, hyperedge_emb, h, X):
    h32 = h.astype(jnp.int32)
    node_emb = lax.reduce_precision(node_emb, 8, 7)
    hyperedge_emb = lax.reduce_precision(hyperedge_emb, 8, 7)
    xr = (X.astype(jnp.int32)
          .reshape(B // BLK, BLK, 2, HALF)
          .transpose(0, 2, 1, 3)
          .reshape(B // BLK * 2, 2, 128))
    out = jax.nn.sigmoid(_sc_call(node_emb, hyperedge_emb, h32, xr))
    return out.reshape(B, 1)


# speedup vs baseline: 3.2525x; 1.0597x over previous
import functools

import jax
import jax.numpy as jnp
from jax import lax
from jax.experimental import pallas as pl
from jax.experimental.pallas import tpu as pltpu
from jax.experimental.pallas import tpu_sc as plsc

B = 16384
K = 32
D = 128
NC = 2
NS = 16
NW = NC * NS
BPW = B // NW
BLK = 16
NBLK = BPW // BLK
HALF = 16


def _rne_bf16(v):
    u = plsc.bitcast(v, jnp.int32)
    t = lax.shift_right_logical(u, 16) & 1
    r = (u + 32767 + t) & jnp.int32(-65536)
    return plsc.bitcast(r, jnp.float32)


def _tree(vals, op):
    vals = list(vals)
    while len(vals) > 1:
        vals = [op(vals[i], vals[i + 1]) if i + 1 < len(vals) else vals[i]
                for i in range(0, len(vals), 2)]
    return vals[0]


def _sc_body(node_hbm, hyper_hbm, h_hbm, xr_hbm, out_hbm,
             xidx, hidx, emb, src, outb, tbuf, mrows, mbuf, se0, se1, ss):
    wid = lax.axis_index("s") * NC + lax.axis_index("c")
    base_blk = wid * NBLK
    lanes = lax.iota(jnp.int32, 16)

    pltpu.sync_copy(xr_hbm.at[pl.ds(base_blk * 2, 2 * NBLK)], xidx)
    pltpu.sync_copy(h_hbm.at[pl.ds(wid * BPW, BPW)], hidx)

    def issue_emb(tloc, hf):
        sem = se0 if hf == 0 else se1
        pltpu.async_copy(node_hbm.at[xidx.at[tloc, 0]],
                         emb.at[hf, pl.ds(0, 128)], sem)
        pltpu.async_copy(node_hbm.at[xidx.at[tloc, 1]],
                         emb.at[hf, pl.ds(128, 128)], sem)

    def wait_emb(tloc, hf):
        sem = se0 if hf == 0 else se1
        pltpu.make_async_copy(node_hbm.at[xidx.at[tloc, 0]],
                              emb.at[hf, pl.ds(0, 128)], sem).wait()
        pltpu.make_async_copy(node_hbm.at[xidx.at[tloc, 1]],
                              emb.at[hf, pl.ds(128, 128)], sem).wait()

    def issue_src(blk, pb):
        pltpu.async_copy(hyper_hbm.at[hidx.at[pl.ds(blk * BLK, BLK)]],
                         src.at[pb], ss)

    def wait_src(blk, pb):
        pltpu.make_async_copy(hyper_hbm.at[hidx.at[pl.ds(blk * BLK, BLK)]],
                              src.at[pb], ss).wait()

    def compute_half(hf, pb):
        @pl.loop(0, BLK)
        def cloop(c):
            sv = [_rne_bf16(src[pb, c, pl.ds(i * 16, 16)]) for i in range(8)]
            accs = []
            for j in range(HALF):
                row = c * HALF + j
                p = [sv[i] * emb[hf, row, pl.ds(i * 16, 16)] for i in range(8)]
                accs.append(_tree(p, lambda a, b: a + b))
            for j in range(HALF):
                tbuf[j, pl.ds(0, 16)] = accs[j]
            cols = [plsc.load_gather(tbuf, [lanes, jnp.full((16,), l, jnp.int32)])
                    for l in range(16)]
            mrows[c, pl.ds(0, 16)] = _tree(cols, lambda a, b: a + b)
        cols = [plsc.load_gather(mrows, [lanes, jnp.full((16,), l, jnp.int32)])
                for l in range(16)]
        return _tree(cols, jnp.minimum)

    issue_src(0, 0)
    issue_emb(0, 0)

    @pl.loop(0, NBLK // 2)
    def superblock(sb):
        for pb in (0, 1):
            blk = sb * 2 + pb
            tloc = blk * 2
            issue_emb(tloc + 1, 1)
            wait_src(blk, pb)
            wait_emb(tloc, 0)
            mbuf[pl.ds(0, BLK)] = compute_half(0, pb)
            if pb == 0:
                issue_emb(tloc + 2, 0)
                issue_src(blk + 1, 1)
            else:
                @pl.when(sb + 1 < NBLK // 2)
                def _():
                    issue_emb(tloc + 2, 0)
                    issue_src(blk + 1, 0)
            wait_emb(tloc + 1, 1)
            m = jnp.minimum(compute_half(1, pb), mbuf[pl.ds(0, BLK)])
            outb[pl.ds(blk * BLK, BLK)] = m

    pltpu.sync_copy(outb, out_hbm.at[pl.ds(wid * BPW, BPW)])


@jax.jit
def _sc_call(node_emb, hyperedge_emb, h32, xr):
    mesh = plsc.VectorSubcoreMesh(core_axis_name="c", subcore_axis_name="s")
    return pl.kernel(
        _sc_body,
        out_type=jax.ShapeDtypeStruct((B,), jnp.float32),
        mesh=mesh,
        compiler_params=pltpu.CompilerParams(needs_layout_passes=False),
        scratch_types=[
            pltpu.VMEM((2 * NBLK, 2, 128), jnp.int32),
            pltpu.VMEM((BPW,), jnp.int32),
            pltpu.VMEM((2, 256, 128), jnp.float32),
            pltpu.VMEM((2, BLK, 128), jnp.float32),
            pltpu.VMEM((BPW,), jnp.float32),
            pltpu.VMEM((HALF, 17), jnp.float32),
            pltpu.VMEM((BLK, 17), jnp.float32),
            pltpu.VMEM((BLK,), jnp.float32),
            pltpu.SemaphoreType.DMA,
            pltpu.SemaphoreType.DMA,
            pltpu.SemaphoreType.DMA,
        ],
    )(node_emb, hyperedge_emb, h32, xr)


def kernel(node_emb, hyperedge_emb, h, X):
    h32 = h.astype(jnp.int32)
    node_emb = lax.reduce_precision(node_emb, 8, 7)
    xr = (X.astype(jnp.int32)
          .reshape(B // BLK, BLK, 2, HALF)
          .transpose(0, 2, 1, 3)
          .reshape(B // BLK * 2, 2, 128))
    out = jax.nn.sigmoid(_sc_call(node_emb, hyperedge_emb, h32, xr))
    return out.reshape(B, 1)
